# baseline (speedup 1.0000x reference)
import functools

import jax
import jax.numpy as jnp
from jax import lax
from jax.experimental import pallas as pl
from jax.experimental.pallas import tpu as pltpu
from jax.experimental.pallas import tpu_sc as plsc

VOCAB = 1000
EMBED = 128
BATCH = 4096
SEQ = 200
N = BATCH * SEQ

NC = 2
NS = 16
NW = NC * NS
B_PER_W = N // NW
CHUNK = 128
K = 2
SET = K * CHUNK
BODY = 2 * SET
NB = B_PER_W // BODY
W_BLK = 64


@functools.partial(
    pl.kernel,
    out_type=jax.ShapeDtypeStruct((N, EMBED), jnp.float32),
    mesh=plsc.VectorSubcoreMesh(core_axis_name="c", subcore_axis_name="s"),
    scratch_types=(
        [pltpu.VMEM((B_PER_W,), jnp.int32)]
        + [pltpu.VMEM_SHARED((VOCAB, EMBED), jnp.float32)]
        + [pltpu.VMEM((SET, EMBED), jnp.float32) for _ in range(2)]
        + [pltpu.SemaphoreType.DMA for _ in range(2 * K + 2)]
    ),
)
def _embed_lookup(x_hbm, w_hbm, out_hbm, idx_v, w_sh, rows_a, rows_b, *sems):
    gsem_a = sems[:K]
    gsem_b = sems[K:2 * K]
    ssem_a, ssem_b = sems[2 * K], sems[2 * K + 1]

    sid = lax.axis_index("s")
    wid = sid * NC + lax.axis_index("c")
    base = wid * B_PER_W

    @pl.when(sid < NS - 1)
    def _():
        pltpu.sync_copy(
            w_hbm.at[pl.ds(sid * W_BLK, W_BLK)],
            w_sh.at[pl.ds(sid * W_BLK, W_BLK)],
        )

    @pl.when(sid == NS - 1)
    def _():
        pltpu.sync_copy(
            w_hbm.at[pl.ds((NS - 1) * W_BLK, VOCAB - (NS - 1) * W_BLK)],
            w_sh.at[pl.ds((NS - 1) * W_BLK, VOCAB - (NS - 1) * W_BLK)],
        )

    pltpu.sync_copy(x_hbm.at[pl.ds(base, B_PER_W)], idx_v)
    plsc.subcore_barrier()

    def idx_slice(local_off):
        return idx_v.at[pl.ds(local_off, CHUNK)]

    def step(i, carry):
        loc_a = i * BODY
        loc_b = loc_a + SET
        off_a = base + loc_a
        off_b = base + loc_b

        @pl.when(i > 0)
        def _():
            pltpu.make_async_copy(
                rows_a, out_hbm.at[pl.ds(off_a, SET)], ssem_a
            ).wait()
        for j in range(K):
            pltpu.async_copy(
                w_sh.at[idx_slice(loc_a + j * CHUNK)],
                rows_a.at[pl.ds(j * CHUNK, CHUNK)], gsem_a[j]
            )

        @pl.when(i > 0)
        def _():
            pltpu.make_async_copy(
                rows_b, out_hbm.at[pl.ds(off_b, SET)], ssem_b
            ).wait()
        for j in range(K):
            pltpu.async_copy(
                w_sh.at[idx_slice(loc_b + j * CHUNK)],
                rows_b.at[pl.ds(j * CHUNK, CHUNK)], gsem_b[j]
            )

        for j in range(K):
            pltpu.make_async_copy(
                w_sh.at[idx_slice(loc_a + j * CHUNK)],
                rows_a.at[pl.ds(j * CHUNK, CHUNK)], gsem_a[j]
            ).wait()
        pltpu.async_copy(rows_a, out_hbm.at[pl.ds(off_a, SET)], ssem_a)

        for j in range(K):
            pltpu.make_async_copy(
                w_sh.at[idx_slice(loc_b + j * CHUNK)],
                rows_b.at[pl.ds(j * CHUNK, CHUNK)], gsem_b[j]
            ).wait()
        pltpu.async_copy(rows_b, out_hbm.at[pl.ds(off_b, SET)], ssem_b)
        return carry

    lax.fori_loop(0, NB, step, 0)

    pltpu.make_async_copy(rows_a, out_hbm.at[pl.ds(base, SET)], ssem_a).wait()
    pltpu.make_async_copy(rows_b, out_hbm.at[pl.ds(base, SET)], ssem_b).wait()


def kernel(x, weight):
    xf = x.reshape(N).astype(jnp.int32)
    out = _embed_lookup(xf, weight)
    return out.reshape(BATCH, SEQ, EMBED)
